# Initial kernel scaffold; baseline (speedup 1.0000x reference)
import jax
import jax.numpy as jnp
from jax.experimental import pallas as pl


def kernel(tokens, wte_weight, learned_embedding):
    raise NotImplementedError("write your pallas kernel here")


# speedup vs baseline: 4.5807x; 4.5807x over previous
import functools

import jax
import jax.numpy as jnp
from jax import lax
from jax.experimental import pallas as pl
from jax.experimental.pallas import tpu as pltpu
from jax.experimental.pallas import tpu_sc as plsc


def kernel(tokens, wte_weight, learned_embedding):
    B, S = tokens.shape
    V, D = wte_weight.shape
    P = learned_embedding.shape[0]
    tokens = tokens.astype(jnp.int32)

    info = plsc.get_sparse_core_info()
    NC, NS = info.num_cores, info.num_subcores
    NW = NC * NS
    assert B % NW == 0
    rows_per_w = B // NW

    mesh = plsc.VectorSubcoreMesh(core_axis_name="c", subcore_axis_name="s")

    @functools.partial(
        pl.kernel,
        mesh=mesh,
        out_type=jax.ShapeDtypeStruct((B, S, D), jnp.float32),
        scratch_types=[
            pltpu.VMEM((S,), jnp.int32),
            pltpu.VMEM((S, D), jnp.float32),
            pltpu.VMEM((P, D), jnp.float32),
            pltpu.SemaphoreType.DMA,
        ],
        compiler_params=pltpu.CompilerParams(use_tc_tiling_on_sc=False),
    )
    def run(tokens_hbm, wte_hbm, learned_hbm, out_hbm, idx_v, buf_v, le_v, sem):
        wid = lax.axis_index("s") * NC + lax.axis_index("c")
        base = wid * rows_per_w
        pltpu.sync_copy(learned_hbm, le_v)

        def body(i, carry):
            b = base + i
            pltpu.sync_copy(tokens_hbm.at[b], idx_v)
            h1 = pltpu.async_copy(
                wte_hbm.at[idx_v.at[pl.ds(0, 128)]], buf_v.at[pl.ds(0, 128)], sem
            )
            h2 = pltpu.async_copy(
                wte_hbm.at[idx_v.at[pl.ds(128, S - 128)]],
                buf_v.at[pl.ds(128, S - 128)],
                sem,
            )
            h1.wait()
            h2.wait()
            for r in range(P):
                for c in range(D // 16):
                    buf_v[r, pl.ds(c * 16, 16)] = le_v[r, pl.ds(c * 16, 16)]
            pltpu.sync_copy(buf_v, out_hbm.at[b])
            return carry

        lax.fori_loop(0, rows_per_w, body, 0)

    return run(tokens, wte_weight, learned_embedding)


# speedup vs baseline: 5.5564x; 1.2130x over previous
import functools

import jax
import jax.numpy as jnp
from jax import lax
from jax.experimental import pallas as pl
from jax.experimental.pallas import tpu as pltpu
from jax.experimental.pallas import tpu_sc as plsc

_R = 2
_NBUF = 4


def kernel(tokens, wte_weight, learned_embedding):
    B, S = tokens.shape
    V, D = wte_weight.shape
    P = learned_embedding.shape[0]
    G = S - P

    tok = jnp.pad(tokens.astype(jnp.int32)[:, P:], ((0, 0), (0, P))).reshape(-1)

    info = plsc.get_sparse_core_info()
    NC, NS = info.num_cores, info.num_subcores
    NW = NC * NS
    assert B % (NW * _R) == 0
    n_chunks = B // (NW * _R)
    assert n_chunks % _NBUF == 0 and n_chunks // _NBUF >= 3
    CROWS = _R * S

    mesh = plsc.VectorSubcoreMesh(core_axis_name="c", subcore_axis_name="s")

    @functools.partial(
        pl.kernel,
        mesh=mesh,
        out_type=jax.ShapeDtypeStruct((B * S, D), jnp.float32),
        scratch_types=(
            [pltpu.VMEM((_R * S,), jnp.int32) for _ in range(_NBUF)]
            + [pltpu.VMEM((_R * S, D), jnp.float32) for _ in range(_NBUF)]
            + [pltpu.SemaphoreType.DMA((_NBUF,)), pltpu.SemaphoreType.DMA((_NBUF,))]
        ),
        compiler_params=pltpu.CompilerParams(use_tc_tiling_on_sc=False),
    )
    def run(tok_hbm, wte_hbm, learned_hbm, out_hbm, *scratch):
        idx_v = scratch[:_NBUF]
        buf_v = scratch[_NBUF : 2 * _NBUF]
        sem_g, sem_o = scratch[2 * _NBUF], scratch[2 * _NBUF + 1]
        wid = lax.axis_index("s") * NC + lax.axis_index("c")
        chunk0 = wid * n_chunks

        for p in range(_NBUF):
            for r in range(_R):
                pltpu.sync_copy(learned_hbm, buf_v[p].at[pl.ds(r * S, P)])

        def gather_descs(p):
            descs = []
            for r in range(_R):
                for off, ln in ((0, 128), (128, G - 128)):
                    descs.append(
                        pltpu.make_async_copy(
                            wte_hbm.at[idx_v[p].at[pl.ds(r * S + off, ln)]],
                            buf_v[p].at[pl.ds(r * S + P + off, ln)],
                            sem_g.at[p],
                        )
                    )
            return descs

        def write_desc(c, p):
            return pltpu.make_async_copy(
                buf_v[p], out_hbm.at[pl.ds((chunk0 + c) * CROWS, CROWS)], sem_o.at[p]
            )

        def wait_gather(c, p):
            for d in gather_descs(p):
                d.wait()

        def start_chunk(c, p):
            pltpu.sync_copy(
                tok_hbm.at[pl.ds((chunk0 + c) * CROWS, CROWS)], idx_v[p]
            )
            for d in gather_descs(p):
                d.start()

        start_chunk(0, 0)
        start_chunk(1, 1)

        def phase(c, p, wait_prev_write, start_next):
            wait_gather(c, p)
            write_desc(c, p).start()
            if wait_prev_write:
                q = (p + _NBUF - 1) % _NBUF
                write_desc(c - 1, q).wait()
            if start_next:
                start_chunk(c + 2, (p + 2) % _NBUF)

        phase(0, 0, False, True)
        for p in range(1, _NBUF):
            phase(p, p, True, True)

        def body(j, carry):
            c0 = j * _NBUF
            for p in range(_NBUF):
                phase(c0 + p, p, True, True)
            return carry

        lax.fori_loop(1, n_chunks // _NBUF - 1, body, 0)

        c0 = n_chunks - _NBUF
        for p in range(_NBUF):
            phase(c0 + p, p, True, c0 + p + 2 < n_chunks)
        write_desc(n_chunks - 1, _NBUF - 1).wait()

    out = run(tok, wte_weight, learned_embedding)
    return out.reshape(B, S, D)
